# baseline (speedup 1.0000x reference)
import functools
import jax
import jax.numpy as jnp
from jax import lax
from jax.experimental import pallas as pl
from jax.experimental.pallas import tpu as pltpu
from jax.experimental.pallas import tpu_sc as plsc

_B, _C, _HW = 16, 384, 4096
_ROWS = _B * _C
_NC, _NS = 2, 16
_NW = _NC * _NS
_RPW = _ROWS // _NW
_K = 8


def kernel(input):
    x = input.reshape(_ROWS, _HW)
    mesh = plsc.VectorSubcoreMesh(core_axis_name="c", subcore_axis_name="s")

    @functools.partial(
        pl.kernel,
        mesh=mesh,
        out_type=jax.ShapeDtypeStruct((_ROWS, _HW), jnp.float32),
        scratch_types=[pltpu.SemaphoreType.DMA],
    )
    def k(in_hbm, out_hbm, sem):
        wid = lax.axis_index("s") * _NC + lax.axis_index("c")
        dst0 = wid * _RPW
        b = wid // 2
        c0 = (wid % 2) * _RPW
        src0 = b * _C + (_C - 1) - c0

        def group(g, carry):
            base = g * _K
            for i in range(_K):
                pltpu.make_async_copy(
                    in_hbm.at[src0 - (base + i)],
                    out_hbm.at[dst0 + base + i],
                    sem,
                ).start()
            for i in range(_K):
                pltpu.make_async_copy(
                    in_hbm.at[src0 - (base + i)],
                    out_hbm.at[dst0 + base + i],
                    sem,
                ).wait()
            return carry

        lax.fori_loop(0, _RPW // _K, group, 0)

    return k(x).reshape(_B, _C, 64, 64)


# speedup vs baseline: 6.2783x; 6.2783x over previous
import functools
import jax
import jax.numpy as jnp
from jax import lax
from jax.experimental import pallas as pl
from jax.experimental.pallas import tpu as pltpu
from jax.experimental.pallas import tpu_sc as plsc

_B, _C = 16, 384
_ROWS = _B * _C
_NC, _NS = 2, 16
_NW = _NC * _NS
_RPW = _ROWS // _NW
_CK = 12
_NCHUNK = _RPW // _CK


def kernel(input):
    x = input.reshape(_ROWS, 32, 128)
    mesh = plsc.VectorSubcoreMesh(core_axis_name="c", subcore_axis_name="s")

    @functools.partial(
        pl.kernel,
        mesh=mesh,
        out_type=jax.ShapeDtypeStruct((_ROWS, 32, 128), jnp.float32),
        scratch_types=[
            pltpu.VMEM((2, _CK, 32, 128), jnp.float32),
            pltpu.SemaphoreType.DMA,
            pltpu.SemaphoreType.DMA,
        ],
    )
    def k(in_hbm, out_hbm, buf, in_sem, out_sem):
        wid = lax.axis_index("s") * _NC + lax.axis_index("c")
        dst0 = wid * _RPW
        b = wid // 2
        c0 = (wid % 2) * _RPW
        src0 = b * _C + (_C - 1) - c0

        def in_copy(t, p):
            lo = src0 - t * _CK - (_CK - 1)
            return pltpu.make_async_copy(in_hbm.at[pl.ds(lo, _CK)], buf.at[p], in_sem)

        def out_copy(t, p, i):
            d = dst0 + t * _CK + (_CK - 1 - i)
            return pltpu.make_async_copy(buf.at[p, i], out_hbm.at[d], out_sem)

        for t in range(_NCHUNK):
            p = t % 2
            if t >= 2:
                for i in range(_CK):
                    out_copy(t - 2, p, i).wait()
            in_copy(t, p).start()
            in_copy(t, p).wait()
            for i in range(_CK):
                out_copy(t, p, i).start()
        for t in (_NCHUNK - 2, _NCHUNK - 1):
            for i in range(_CK):
                out_copy(t, t % 2, i).wait()

    return k(x).reshape(_B, _C, 64, 64)


# speedup vs baseline: 15.4790x; 2.4655x over previous
import functools
import jax
import jax.numpy as jnp
from jax import lax
from jax.experimental import pallas as pl
from jax.experimental.pallas import tpu as pltpu
from jax.experimental.pallas import tpu_sc as plsc

_C = 384
_NROW = 16 * 64 * 64
_NC, _NS = 2, 16
_NW = _NC * _NS
_RPW = _NROW // _NW
_CK = 64
_NCHUNK = _RPW // _CK
_NV = _C // 16


def kernel(input):
    x = jnp.transpose(input, (0, 2, 3, 1)).reshape(_NROW, _C)
    mesh = plsc.VectorSubcoreMesh(core_axis_name="c", subcore_axis_name="s")

    @functools.partial(
        pl.kernel,
        mesh=mesh,
        out_type=jax.ShapeDtypeStruct((_NROW, _C), jnp.float32),
        scratch_types=[
            pltpu.VMEM((2, _CK, _C), jnp.float32),
            pltpu.VMEM((2, _CK, _C), jnp.float32),
            pltpu.SemaphoreType.DMA,
            pltpu.SemaphoreType.DMA,
            pltpu.SemaphoreType.DMA,
        ],
    )
    def k(in_hbm, out_hbm, ibuf, obuf, in_sem, out_sem0, out_sem1):
        out_sems = (out_sem0, out_sem1)
        wid = lax.axis_index("s") * _NC + lax.axis_index("c")
        base = wid * _RPW

        def in_copy(t, p):
            return pltpu.make_async_copy(
                in_hbm.at[pl.ds(base + t * _CK, _CK)], ibuf.at[p], in_sem
            )

        def out_copy(t, p):
            return pltpu.make_async_copy(
                obuf.at[p], out_hbm.at[pl.ds(base + t * _CK, _CK)], out_sems[p]
            )

        def reverse_chunk(p):
            def row(i, carry):
                for j in range(_NV):
                    v = ibuf[p, i, pl.ds(16 * j, 16)]
                    obuf[p, i, pl.ds(16 * (_NV - 1 - j), 16)] = lax.rev(v, (0,))
                return carry
            lax.fori_loop(0, _CK, row, 0)

        for t in range(_NCHUNK):
            p = t % 2
            in_copy(t, p).start()
            in_copy(t, p).wait()
            if t >= 2:
                out_copy(t - 2, p).wait()
            reverse_chunk(p)
            out_copy(t, p).start()
        out_copy(_NCHUNK - 2, 0).wait()
        out_copy(_NCHUNK - 1, 1).wait()

    y = k(x)
    return jnp.transpose(y.reshape(16, 64, 64, _C), (0, 3, 1, 2))
